# baseline (speedup 1.0000x reference)
import functools
import math

import jax
import jax.numpy as jnp
from jax import lax
from jax.experimental import pallas as pl
from jax.experimental.pallas import tpu as pltpu
from jax.experimental.pallas import tpu_sc as plsc

_GAMMAS = [2.7, 2.1, 2.4, 2.0, 3.0, 2.9, 3.0, 2.5, 2.1, 2.6, 2.0, 2.1, 2.7, 2.4, 2.2]
_B, _C, _H, _W, _D, _K = 4, 15, 128, 128, 2, 32
_NPTS = _B * _K
_EPS = 1e-12
_TINY = 1e-30


def _safe_pow(x, g):
    return jnp.exp(g * jnp.log(jnp.maximum(x, _TINY)))


_ATAN_COEFFS = [0.9999999937538815, -0.33333137974717497, 0.19993694319379748,
                -0.14211106054466893, 0.10667486902150858, -0.07556900202159014,
                0.043278241738805345, -0.016413190395028338, 0.0029327619363945373]


def _atan_pos(x):
    inv = x > 1.0
    t = jnp.where(inv, 1.0 / jnp.maximum(x, _TINY), x)
    t2 = t * t
    p = _ATAN_COEFFS[-1]
    for cf in _ATAN_COEFFS[-2::-1]:
        p = p * t2 + cf
    p = t * p
    return jnp.where(inv, 0.5 * math.pi - p, p)


def _tc_dense_body(pred_ref, gt_ref, out_ref, acc):
    i = pl.program_id(0)

    @pl.when(i == 0)
    def _init():
        acc[0] = 0.0
        acc[1] = 0.0

    accv = jnp.zeros((_H, _W), jnp.float32)
    nposv = jnp.zeros((_H, _W), jnp.float32)
    for c in range(_C):
        g = _GAMMAS[c]
        pb = pred_ref[c]
        gb = gt_ref[c]
        lp = jnp.log(pb + _EPS)
        l1p = jnp.log((1.0 - pb) + _EPS)
        posf = gb == 1.0
        omg = 1.0 - gb
        negw = (omg * omg) * (omg * omg)
        pos_term = lp * jnp.exp(g * l1p)
        neg_term = l1p * jnp.exp(g * lp) * negw
        accv = accv + g * jnp.where(posf, pos_term, neg_term)
        nposv = nposv + jnp.where(posf, 1.0, 0.0)
    acc[0] = acc[0] - 0.5 * jnp.sum(accv)
    acc[1] = acc[1] + jnp.sum(nposv)

    @pl.when(i == _B - 1)
    def _done():
        out_ref[0, 0] = acc[0]
        out_ref[0, 1] = acc[1]


def _tc_final_body(smalls_ref, partial_ref, out_ref):
    pre0 = smalls_ref[0:1, :]
    pre1 = smalls_ref[1:2, :]
    tgt0 = smalls_ref[2:3, :]
    tgt1 = smalls_ref[3:4, :]
    maskf = smalls_ref[4:5, :]
    c0f = smalls_ref[5:6, :]
    c1f = smalls_ref[6:7, :]
    c2f = smalls_ref[7:8, :]
    x = smalls_ref[8:9, :]
    g = smalls_ref[9:10, :]

    def _sl1(d):
        ad = jnp.abs(d)
        return jnp.where(ad < 1.0, 0.5 * d * d, ad - 0.5)

    li = 0.5 * (_sl1(pre0 - tgt0) + _sl1(pre1 - tgt1))
    iota = jax.lax.broadcasted_iota(jnp.int32, (1, _NPTS), 1)
    bf = jnp.floor(iota.astype(jnp.float32) / float(_K))

    masked_idx = jnp.where(maskf > 0.5, iota, -1)
    last = jnp.max(masked_idx)
    loss0 = jnp.sum(jnp.where(iota == last, li, 0.0))

    factor = _atan_pos(li) * (2.0 / math.pi)
    factor = jnp.where(maskf > 0.5, factor, 1.0)
    logf = jnp.log(jnp.maximum(factor, 1e-37))

    Ff = ((bf * 15.0 + c0f) * 15.0 + c1f) * 15.0 + c2f
    Fcol = jnp.transpose(Ff, (1, 0))
    same = Fcol == jnp.broadcast_to(Ff, (_NPTS, _NPTS))
    lsum_col = jnp.sum(
        jnp.where(same, jnp.broadcast_to(logf, (_NPTS, _NPTS)), 0.0),
        axis=1, keepdims=True)
    lsum = jnp.transpose(lsum_col, (1, 0))
    m = jnp.exp(lsum)
    jrow = jnp.broadcast_to(iota, (_NPTS, _NPTS))
    minj_col = jnp.min(jnp.where(same, jrow, _NPTS), axis=1, keepdims=True)
    minj = jnp.transpose(minj_col, (1, 0))
    first = (minj == iota).astype(jnp.float32)

    gamma_pt = jnp.zeros_like(c0f)
    for j, gv in enumerate(_GAMMAS):
        gamma_pt = jnp.where(c0f == float(j), gv, gamma_pt)

    posp = (g == 1.0).astype(jnp.float32)
    negp = (g < 1.0).astype(jnp.float32)
    omgp = 1.0 - g
    negwp = (omgp * omgp) * (omgp * omgp)

    def _floss(xv):
        pt = jnp.log(xv + _EPS) * _safe_pow(1.0 - xv, gamma_pt) * posp
        nt = jnp.log(1.0 - xv + _EPS) * _safe_pow(xv, gamma_pt) * negwp * negp
        return pt + nt

    delta = _floss(x * m) - _floss(x)
    corr = jnp.sum(first * (-0.5 * gamma_pt) * delta)

    total = partial_ref[0, 0] + loss0 + corr
    npos = partial_ref[0, 1]
    out_ref[0, 0] = jnp.where(npos == 0.0, total, total / npos)


def _tc_dense_call(pred2, gt2):
    return pl.pallas_call(
        _tc_dense_body,
        grid=(_B,),
        in_specs=[
            pl.BlockSpec((_C, _H, _W), lambda i: (i, 0, 0)),
            pl.BlockSpec((_C, _H, _W), lambda i: (i, 0, 0)),
        ],
        out_specs=pl.BlockSpec((1, 2), lambda i: (0, 0), memory_space=pltpu.SMEM),
        out_shape=jax.ShapeDtypeStruct((1, 2), jnp.float32),
        scratch_shapes=[pltpu.SMEM((2,), jnp.float32)],
    )(pred2, gt2)


def _tc_final_call(smalls, partial):
    return pl.pallas_call(
        _tc_final_body,
        in_specs=[
            pl.BlockSpec(memory_space=pltpu.VMEM),
            pl.BlockSpec(memory_space=pltpu.SMEM),
        ],
        out_specs=pl.BlockSpec(memory_space=pltpu.SMEM),
        out_shape=jax.ShapeDtypeStruct((1, 1), jnp.float32),
    )(smalls, partial)


def _sc_body(outf, predf, gtf, ind_h, inde_h, mask_h, tgt_h, smalls,
             ivec, fbuf, idxm, out_v, sem_a, sem_b, sem_c, sem_g):
    cid = lax.axis_index("c")
    s = lax.axis_index("s")
    k16a = jax.lax.broadcasted_iota(jnp.int32, (16,), 0)

    @pl.when((cid == 0) & (s == 0))
    def _go():
        h1 = pltpu.async_copy(ind_h, ivec.at[pl.ds(0, 128)], sem_a)
        h2 = pltpu.async_copy(inde_h, ivec.at[pl.ds(128, 384)], sem_b)
        h3 = pltpu.async_copy(mask_h, ivec.at[pl.ds(512, 128)], sem_c)
        h4 = pltpu.async_copy(tgt_h, fbuf, sem_c)

        h1.wait()
        for gi in range(8):
            p16 = k16a + gi * 16
            sl = pl.ds(gi * 16, 16)
            bb = p16 // _K
            indv = plsc.load_gather(ivec, [p16])
            idxm[0, sl] = bb * (_D * _H * _W) + indv
            idxm[1, sl] = bb * (_D * _H * _W) + (_H * _W) + indv
        g0 = pltpu.async_copy(outf.at[idxm.at[0]], out_v.at[0], sem_g)
        g1 = pltpu.async_copy(outf.at[idxm.at[1]], out_v.at[1], sem_g)

        h2.wait()
        for gi in range(8):
            p16 = k16a + gi * 16
            sl = pl.ds(gi * 16, 16)
            bb = p16 // _K
            i3 = 3 * p16 + 128
            cc0 = plsc.load_gather(ivec, [i3])
            cc1 = plsc.load_gather(ivec, [i3 + 1])
            cc2 = plsc.load_gather(ivec, [i3 + 2])
            idxm[2, sl] = bb * (_C * _H * _W) + cc0 * (_H * _W) + cc1 * _W + cc2
            out_v[5, sl] = cc0.astype(jnp.float32)
            out_v[6, sl] = cc1.astype(jnp.float32)
            out_v[7, sl] = cc2.astype(jnp.float32)
        g2 = pltpu.async_copy(predf.at[idxm.at[2]], out_v.at[8], sem_g)
        g3 = pltpu.async_copy(gtf.at[idxm.at[2]], out_v.at[9], sem_g)

        h3.wait(); h4.wait()
        for gi in range(8):
            p16 = k16a + gi * 16
            sl = pl.ds(gi * 16, 16)
            out_v[4, sl] = plsc.load_gather(ivec, [512 + p16]).astype(jnp.float32)
            t2 = 2 * p16
            out_v[2, sl] = plsc.load_gather(fbuf, [t2])
            out_v[3, sl] = plsc.load_gather(fbuf, [t2 + 1])

        g0.wait(); g1.wait(); g2.wait(); g3.wait()
        pltpu.sync_copy(out_v, smalls)


def _sc_call(outf, predf, gtf, ind, inde_flat, mask, tgt_flat):
    mesh = plsc.VectorSubcoreMesh(core_axis_name="c", subcore_axis_name="s",
                                  num_cores=1, num_subcores=1)
    fn = functools.partial(
        pl.kernel,
        mesh=mesh,
        compiler_params=pltpu.CompilerParams(needs_layout_passes=False),
        out_type=jax.ShapeDtypeStruct((16, _NPTS), jnp.float32),
        scratch_types=[
            pltpu.VMEM((640,), jnp.int32),
            pltpu.VMEM((256,), jnp.float32),
            pltpu.VMEM((4, _NPTS), jnp.int32),
            pltpu.VMEM((16, _NPTS), jnp.float32),
            pltpu.SemaphoreType.DMA,
            pltpu.SemaphoreType.DMA,
            pltpu.SemaphoreType.DMA,
            pltpu.SemaphoreType.DMA,
        ],
    )(_sc_body)
    return fn(outf, predf, gtf, ind.reshape(_B * _K), inde_flat.reshape(_B * 96),
              mask.reshape(_B * _K), tgt_flat.reshape(_B * 64))


def kernel(pred, gt, output, mask, ind, target, inde):
    outf = output.reshape(_B * _D * _H * _W)
    predf = pred.reshape(_B * _C * _H * _W)
    gtf = gt.reshape(_B * _C * _H * _W)
    smalls = _sc_call(outf, predf, gtf, ind, inde, mask, target)
    partial = _tc_dense_call(pred.reshape(_B * _C, _H, _W),
                             gt.reshape(_B * _C, _H, _W))
    res = _tc_final_call(smalls, partial)
    return res.reshape(())


# speedup vs baseline: 1.0045x; 1.0045x over previous
import functools
import math

import jax
import jax.numpy as jnp
from jax import lax
from jax.experimental import pallas as pl
from jax.experimental.pallas import tpu as pltpu
from jax.experimental.pallas import tpu_sc as plsc

_GAMMAS = [2.7, 2.1, 2.4, 2.0, 3.0, 2.9, 3.0, 2.5, 2.1, 2.6, 2.0, 2.1, 2.7, 2.4, 2.2]
_B, _C, _H, _W, _D, _K = 4, 15, 128, 128, 2, 32
_NPTS = _B * _K
_EPS = 1e-12
_TINY = 1e-30


def _safe_pow(x, g):
    return jnp.exp(g * jnp.log(jnp.maximum(x, _TINY)))


_ATAN_COEFFS = [0.9999999937538815, -0.33333137974717497, 0.19993694319379748,
                -0.14211106054466893, 0.10667486902150858, -0.07556900202159014,
                0.043278241738805345, -0.016413190395028338, 0.0029327619363945373]


def _atan_pos(x):
    inv = x > 1.0
    t = jnp.where(inv, 1.0 / jnp.maximum(x, _TINY), x)
    t2 = t * t
    p = _ATAN_COEFFS[-1]
    for cf in _ATAN_COEFFS[-2::-1]:
        p = p * t2 + cf
    p = t * p
    return jnp.where(inv, 0.5 * math.pi - p, p)


def _tc_dense_body(pred_ref, gt_ref, out_ref, acc):
    i = pl.program_id(0)

    @pl.when(i == 0)
    def _init():
        acc[0] = 0.0
        acc[1] = 0.0

    accv = jnp.zeros((_H, _W), jnp.float32)
    nposv = jnp.zeros((_H, _W), jnp.float32)
    for c in range(_C):
        g = _GAMMAS[c]
        pb = pred_ref[c]
        gb = gt_ref[c]
        lp = jnp.log(pb + _EPS)
        l1p = jnp.log((1.0 - pb) + _EPS)
        posf = gb == 1.0
        omg = 1.0 - gb
        negw = (omg * omg) * (omg * omg)
        pos_term = lp * jnp.exp(g * l1p)
        neg_term = l1p * jnp.exp(g * lp) * negw
        accv = accv + g * jnp.where(posf, pos_term, neg_term)
        nposv = nposv + jnp.where(posf, 1.0, 0.0)
    acc[0] = acc[0] - 0.5 * jnp.sum(accv)
    acc[1] = acc[1] + jnp.sum(nposv)

    @pl.when(i == _B - 1)
    def _done():
        out_ref[0, 0] = acc[0]
        out_ref[0, 1] = acc[1]


def _tc_final_body(smalls_ref, partial_ref, out_ref):
    pre0 = smalls_ref[0:1, :]
    pre1 = smalls_ref[1:2, :]
    tgt0 = smalls_ref[2:3, :]
    tgt1 = smalls_ref[3:4, :]
    maskf = smalls_ref[4:5, :]
    c0f = smalls_ref[5:6, :]
    c1f = smalls_ref[6:7, :]
    c2f = smalls_ref[7:8, :]
    x = smalls_ref[8:9, :]
    g = smalls_ref[9:10, :]

    def _sl1(d):
        ad = jnp.abs(d)
        return jnp.where(ad < 1.0, 0.5 * d * d, ad - 0.5)

    li = 0.5 * (_sl1(pre0 - tgt0) + _sl1(pre1 - tgt1))
    iota = jax.lax.broadcasted_iota(jnp.int32, (1, _NPTS), 1)
    bf = jnp.floor(iota.astype(jnp.float32) / float(_K))

    masked_idx = jnp.where(maskf > 0.5, iota, -1)
    last = jnp.max(masked_idx)
    loss0 = jnp.sum(jnp.where(iota == last, li, 0.0))

    factor = _atan_pos(li) * (2.0 / math.pi)
    factor = jnp.where(maskf > 0.5, factor, 1.0)
    logf = jnp.log(jnp.maximum(factor, 1e-37))

    Ff = ((bf * 15.0 + c0f) * 15.0 + c1f) * 15.0 + c2f
    Fcol = jnp.transpose(Ff, (1, 0))
    same = Fcol == jnp.broadcast_to(Ff, (_NPTS, _NPTS))
    lsum_col = jnp.sum(
        jnp.where(same, jnp.broadcast_to(logf, (_NPTS, _NPTS)), 0.0),
        axis=1, keepdims=True)
    lsum = jnp.transpose(lsum_col, (1, 0))
    m = jnp.exp(lsum)
    jrow = jnp.broadcast_to(iota, (_NPTS, _NPTS))
    minj_col = jnp.min(jnp.where(same, jrow, _NPTS), axis=1, keepdims=True)
    minj = jnp.transpose(minj_col, (1, 0))
    first = (minj == iota).astype(jnp.float32)

    gamma_pt = jnp.zeros_like(c0f)
    for j, gv in enumerate(_GAMMAS):
        gamma_pt = jnp.where(c0f == float(j), gv, gamma_pt)

    posp = (g == 1.0).astype(jnp.float32)
    negp = (g < 1.0).astype(jnp.float32)
    omgp = 1.0 - g
    negwp = (omgp * omgp) * (omgp * omgp)

    def _floss(xv):
        pt = jnp.log(xv + _EPS) * _safe_pow(1.0 - xv, gamma_pt) * posp
        nt = jnp.log(1.0 - xv + _EPS) * _safe_pow(xv, gamma_pt) * negwp * negp
        return pt + nt

    delta = _floss(x * m) - _floss(x)
    corr = jnp.sum(first * (-0.5 * gamma_pt) * delta)

    total = partial_ref[0, 0] + loss0 + corr
    npos = partial_ref[0, 1]
    out_ref[0, 0] = jnp.where(npos == 0.0, total, total / npos)


def _tc_dense_call(pred2, gt2):
    return pl.pallas_call(
        _tc_dense_body,
        grid=(_B,),
        in_specs=[
            pl.BlockSpec((_C, _H, _W), lambda i: (i, 0, 0)),
            pl.BlockSpec((_C, _H, _W), lambda i: (i, 0, 0)),
        ],
        out_specs=pl.BlockSpec((1, 2), lambda i: (0, 0), memory_space=pltpu.SMEM),
        out_shape=jax.ShapeDtypeStruct((1, 2), jnp.float32),
        scratch_shapes=[pltpu.SMEM((2,), jnp.float32)],
    )(pred2, gt2)


def _tc_final_call(smalls, partial):
    return pl.pallas_call(
        _tc_final_body,
        in_specs=[
            pl.BlockSpec(memory_space=pltpu.VMEM),
            pl.BlockSpec(memory_space=pltpu.SMEM),
        ],
        out_specs=pl.BlockSpec(memory_space=pltpu.SMEM),
        out_shape=jax.ShapeDtypeStruct((1, 1), jnp.float32),
    )(smalls, partial)


def _sc_body(outf, predf, gtf, ind_h, inde_h, mask_h, tgt_h, smalls,
             ivec, fbuf, idxm, out_v, sem_a, sem_b, sem_c, sem_g):
    cid = lax.axis_index("c")
    s = lax.axis_index("s")
    k16a = jax.lax.broadcasted_iota(jnp.int32, (16,), 0)

    @pl.when((cid == 0) & (s == 0))
    def _go():
        h1 = pltpu.async_copy(ind_h, ivec.at[pl.ds(0, 128)], sem_a)
        h2 = pltpu.async_copy(inde_h, ivec.at[pl.ds(128, 384)], sem_b)
        h3 = pltpu.async_copy(mask_h, ivec.at[pl.ds(512, 128)], sem_c)
        h4 = pltpu.async_copy(tgt_h, fbuf, sem_c)

        h1.wait()
        for gi in range(8):
            p16 = k16a + gi * 16
            sl = pl.ds(gi * 16, 16)
            bb = p16 // _K
            indv = plsc.load_gather(ivec, [p16])
            idxm[0, sl] = bb * (_D * _H * _W) + indv
            idxm[1, sl] = bb * (_D * _H * _W) + (_H * _W) + indv
        g0 = pltpu.async_copy(outf.at[idxm.at[0]], out_v.at[0], sem_g)
        g1 = pltpu.async_copy(outf.at[idxm.at[1]], out_v.at[1], sem_g)

        h2.wait()
        for gi in range(8):
            p16 = k16a + gi * 16
            sl = pl.ds(gi * 16, 16)
            bb = p16 // _K
            i3 = 3 * p16 + 128
            cc0 = plsc.load_gather(ivec, [i3])
            cc1 = plsc.load_gather(ivec, [i3 + 1])
            cc2 = plsc.load_gather(ivec, [i3 + 2])
            idxm[2, sl] = bb * (_C * _H * _W) + cc0 * (_H * _W) + cc1 * _W + cc2
            out_v[5, sl] = cc0.astype(jnp.float32)
            out_v[6, sl] = cc1.astype(jnp.float32)
            out_v[7, sl] = cc2.astype(jnp.float32)
        g2 = pltpu.async_copy(predf.at[idxm.at[2]], out_v.at[8], sem_g)
        g3 = pltpu.async_copy(gtf.at[idxm.at[2]], out_v.at[9], sem_g)

        h3.wait(); h4.wait()
        for gi in range(8):
            p16 = k16a + gi * 16
            sl = pl.ds(gi * 16, 16)
            out_v[4, sl] = plsc.load_gather(ivec, [512 + p16]).astype(jnp.float32)
            t2 = 2 * p16
            out_v[2, sl] = plsc.load_gather(fbuf, [t2])
            out_v[3, sl] = plsc.load_gather(fbuf, [t2 + 1])

        g0.wait(); g1.wait(); g2.wait(); g3.wait()
        pltpu.sync_copy(out_v, smalls)


def _sc_call(outf, predf, gtf, ind, inde_flat, mask, tgt_flat):
    mesh = plsc.VectorSubcoreMesh(core_axis_name="c", subcore_axis_name="s",
                                  num_cores=1)
    fn = functools.partial(
        pl.kernel,
        mesh=mesh,
        compiler_params=pltpu.CompilerParams(needs_layout_passes=False),
        out_type=jax.ShapeDtypeStruct((16, _NPTS), jnp.float32),
        scratch_types=[
            pltpu.VMEM((640,), jnp.int32),
            pltpu.VMEM((256,), jnp.float32),
            pltpu.VMEM((4, _NPTS), jnp.int32),
            pltpu.VMEM((16, _NPTS), jnp.float32),
            pltpu.SemaphoreType.DMA,
            pltpu.SemaphoreType.DMA,
            pltpu.SemaphoreType.DMA,
            pltpu.SemaphoreType.DMA,
        ],
    )(_sc_body)
    return fn(outf, predf, gtf, ind.reshape(_B * _K), inde_flat.reshape(_B * 96),
              mask.reshape(_B * _K), tgt_flat.reshape(_B * 64))


def kernel(pred, gt, output, mask, ind, target, inde):
    outf = output.reshape(_B * _D * _H * _W)
    predf = pred.reshape(_B * _C * _H * _W)
    gtf = gt.reshape(_B * _C * _H * _W)
    smalls = _sc_call(outf, predf, gtf, ind, inde, mask, target)
    partial = _tc_dense_call(pred.reshape(_B * _C, _H, _W),
                             gt.reshape(_B * _C, _H, _W))
    res = _tc_final_call(smalls, partial)
    return res.reshape(())
